# Initial kernel scaffold; baseline (speedup 1.0000x reference)
import jax
import jax.numpy as jnp
from jax.experimental import pallas as pl


def kernel(x, edge_index, W1p, b1p, W2p, b2p, W1t, b1t, W2t, b2t):
    raise NotImplementedError("write your pallas kernel here")


# speedup vs baseline: 8.3226x; 8.3226x over previous
import functools

import jax
import jax.numpy as jnp
from jax import lax
from jax.experimental import pallas as pl
from jax.experimental.pallas import tpu as pltpu
from jax.experimental.pallas import tpu_sc as plsc

N = 10000
E = 320000
D = 128

NC = 2
NS = 16
NW = NC * NS
EPW = E // NW
CB = 80
CK = EPW // CB
NIT = 10
RPT = N // NIT

def _sc_aggregate_body(x_hbm, src_hbm, dst_hbm, zrows_hbm, zflat_hbm,
                       y0_hbm, y1_hbm, c0_hbm, c1_hbm,
                       srcv, dstv, rows, ones, acc, dcnt, sem):
    cid = lax.axis_index("c")
    sid = lax.axis_index("s")
    wid = sid * NC + cid
    rs = pl.ds(sid * RPT, RPT)

    @pl.when(sid < NIT)
    def _():
        pltpu.sync_copy(zrows_hbm.at[rs], acc.at[rs])

    @pl.when(sid == 0)
    def _():
        pltpu.sync_copy(zflat_hbm, dcnt)

    pltpu.sync_copy(src_hbm.at[wid], srcv)
    pltpu.sync_copy(dst_hbm.at[wid], dstv)
    for i in range(CB // 16):
        ones[pl.ds(i * 16, 16)] = jnp.full((16,), 1.0, jnp.float32)

    plsc.subcore_barrier()

    def body(j, carry):
        pltpu.async_copy(x_hbm.at[srcv.at[j]], rows, sem).wait()
        pltpu.sync_copy(rows, acc.at[dstv.at[j]], add=True)
        pltpu.sync_copy(ones, dcnt.at[dstv.at[j]], add=True)
        return carry

    lax.fori_loop(0, CK, body, 0)

    plsc.subcore_barrier()

    @pl.when(cid == 0)
    def _():
        @pl.when(sid < NIT)
        def _():
            pltpu.sync_copy(acc.at[rs], y0_hbm.at[rs])

        @pl.when(sid == 0)
        def _():
            pltpu.sync_copy(dcnt, c0_hbm)

    @pl.when(cid == 1)
    def _():
        @pl.when(sid < NIT)
        def _():
            pltpu.sync_copy(acc.at[rs], y1_hbm.at[rs])

        @pl.when(sid == 0)
        def _():
            pltpu.sync_copy(dcnt, c1_hbm)


@functools.cache
def _sc_aggregate():
    mesh = plsc.VectorSubcoreMesh(core_axis_name="c", subcore_axis_name="s")
    return pl.kernel(
        _sc_aggregate_body,
        out_type=[
            jax.ShapeDtypeStruct((N, D), jnp.float32),
            jax.ShapeDtypeStruct((N, D), jnp.float32),
            jax.ShapeDtypeStruct((N,), jnp.float32),
            jax.ShapeDtypeStruct((N,), jnp.float32),
        ],
        mesh=mesh,
        scratch_types=[
            pltpu.VMEM((CK, CB), jnp.int32),
            pltpu.VMEM((CK, CB), jnp.int32),
            pltpu.VMEM((CB, D), jnp.float32),
            pltpu.VMEM((CB,), jnp.float32),
            pltpu.VMEM_SHARED((N, D), jnp.float32),
            pltpu.VMEM_SHARED((N,), jnp.float32),
            pltpu.SemaphoreType.DMA,
        ],
    )


BN = 1000


def _mid_body(y0, y1, c0, c1, w1p, b1p, w1t, b1t, w2p, w2t, z, dinv, mv):
    cnt = c0[...] + c1[...]
    di = 1.0 / jnp.maximum(cnt, 1.0)
    m = cnt * di
    y = (y0[...] + y1[...]) * di
    ap = jnp.dot(y, w1p[...], preferred_element_type=jnp.float32) + m * b1p[...]
    at = jnp.dot(y, w1t[...], preferred_element_type=jnp.float32) + m * b1t[...]
    rp = jnp.maximum(ap, 0.0)
    rt = jnp.maximum(at, 0.0)
    z[...] = (jnp.dot(rt, w2t[...], preferred_element_type=jnp.float32)
              - jnp.dot(rp, w2p[...], preferred_element_type=jnp.float32))
    dinv[...] = di
    mv[...] = m


def _loss_body(u0, u1, dinv, mv, b2p, b2t, out):
    i = pl.program_id(0)
    nb = pl.num_programs(0)
    diff = (u0[...] + u1[...]) * dinv[...] + mv[...] * (b2t[...] - b2p[...])
    part = jnp.sum(diff * diff)
    tot = jnp.where(i == 0, part, out[...] + part)
    out[...] = tot * jnp.where(i == nb - 1, 1.0 / (N * D), 1.0)


def _row_spec(bn, w):
    return pl.BlockSpec((bn, w), lambda i: (i, 0))


def _full_spec(a, b):
    return pl.BlockSpec((a, b), lambda i: (0, 0))


_tc_mid = pl.pallas_call(
    _mid_body,
    grid=(N // BN,),
    in_specs=[
        _row_spec(BN, D), _row_spec(BN, D),
        _row_spec(BN, 1), _row_spec(BN, 1),
        _full_spec(D, D), _full_spec(1, D),
        _full_spec(D, D), _full_spec(1, D),
        _full_spec(D, D), _full_spec(D, D),
    ],
    out_specs=[_row_spec(BN, D), _row_spec(BN, 1), _row_spec(BN, 1)],
    out_shape=[
        jax.ShapeDtypeStruct((N, D), jnp.float32),
        jax.ShapeDtypeStruct((N, 1), jnp.float32),
        jax.ShapeDtypeStruct((N, 1), jnp.float32),
    ],
)

_tc_loss = pl.pallas_call(
    _loss_body,
    grid=(N // BN,),
    in_specs=[
        _row_spec(BN, D), _row_spec(BN, D),
        _row_spec(BN, 1), _row_spec(BN, 1),
        _full_spec(1, D), _full_spec(1, D),
    ],
    out_specs=pl.BlockSpec((1, 1), lambda i: (0, 0)),
    out_shape=jax.ShapeDtypeStruct((1, 1), jnp.float32),
)


def kernel(x, edge_index, W1p, b1p, W2p, b2p, W1t, b1t, W2t, b2t):
    src3 = edge_index[0].reshape(NW, CK, CB)
    dst3 = edge_index[1].reshape(NW, CK, CB)
    zrows = jnp.zeros((N, D), jnp.float32)
    zflat = jnp.zeros((N,), jnp.float32)

    y0, y1, c0, c1 = _sc_aggregate()(x, src3, dst3, zrows, zflat)
    z, dinv, mv = _tc_mid(y0, y1, c0.reshape(N, 1), c1.reshape(N, 1),
                          W1p, b1p.reshape(1, D), W1t, b1t.reshape(1, D),
                          W2p, W2t)
    u0, u1, _, _ = _sc_aggregate()(z, src3, dst3, zrows, zflat)
    loss = _tc_loss(u0, u1, dinv, mv, b2p.reshape(1, D), b2t.reshape(1, D))
    return loss.reshape(())


# speedup vs baseline: 12.0081x; 1.4428x over previous
import functools

import jax
import jax.numpy as jnp
from jax import lax
from jax.experimental import pallas as pl
from jax.experimental.pallas import tpu as pltpu
from jax.experimental.pallas import tpu_sc as plsc

N = 10000
E = 320000
D = 128

NC = 2
NS = 16
NW = NC * NS
EPW = E // NW
CB = 125
CK = EPW // CB
GC = 16
NG = CK // GC
NIT = 10
RPT = N // NIT

def _sc_aggregate_body(x_hbm, src_hbm, dst_hbm, zrows_hbm, zflat_hbm, ones_hbm,
                       y0_hbm, y1_hbm, c0_hbm, c1_hbm,
                       srcv, dstv, rows0, rows1, ones, acc, dcnt, sem):
    cid = lax.axis_index("c")
    sid = lax.axis_index("s")
    wid = sid * NC + cid
    rs = pl.ds(sid * RPT, RPT)

    @pl.when(sid < NIT)
    def _():
        pltpu.sync_copy(zrows_hbm.at[rs], acc.at[rs])

    @pl.when(sid == 0)
    def _():
        pltpu.sync_copy(zflat_hbm, dcnt)

    pltpu.sync_copy(ones_hbm, ones)

    plsc.subcore_barrier()

    def group(g, carry):
        gs = pl.ds(g * GC, GC)
        pltpu.sync_copy(src_hbm.at[wid, gs], srcv)
        pltpu.sync_copy(dst_hbm.at[wid, gs], dstv)
        pltpu.async_copy(x_hbm.at[srcv.at[0]], rows0, sem)

        def pair(p, c2):
            j0 = 2 * p
            j1 = j0 + 1
            pltpu.make_async_copy(x_hbm.at[srcv.at[j0]], rows0, sem).wait()
            pltpu.async_copy(x_hbm.at[srcv.at[j1]], rows1, sem)
            pltpu.sync_copy(rows0, acc.at[dstv.at[j0]], add=True)
            pltpu.sync_copy(ones, dcnt.at[dstv.at[j0]], add=True)
            pltpu.make_async_copy(x_hbm.at[srcv.at[j1]], rows1, sem).wait()

            @pl.when(p < GC // 2 - 1)
            def _():
                pltpu.async_copy(x_hbm.at[srcv.at[j0 + 2]], rows0, sem)

            pltpu.sync_copy(rows1, acc.at[dstv.at[j1]], add=True)
            pltpu.sync_copy(ones, dcnt.at[dstv.at[j1]], add=True)
            return c2

        lax.fori_loop(0, GC // 2, pair, 0)
        return carry

    lax.fori_loop(0, NG, group, 0)

    plsc.subcore_barrier()

    @pl.when(cid == 0)
    def _():
        @pl.when(sid < NIT)
        def _():
            pltpu.sync_copy(acc.at[rs], y0_hbm.at[rs])

        @pl.when(sid == 0)
        def _():
            pltpu.sync_copy(dcnt, c0_hbm)

    @pl.when(cid == 1)
    def _():
        @pl.when(sid < NIT)
        def _():
            pltpu.sync_copy(acc.at[rs], y1_hbm.at[rs])

        @pl.when(sid == 0)
        def _():
            pltpu.sync_copy(dcnt, c1_hbm)


@functools.cache
def _sc_aggregate():
    mesh = plsc.VectorSubcoreMesh(core_axis_name="c", subcore_axis_name="s")
    return pl.kernel(
        _sc_aggregate_body,
        out_type=[
            jax.ShapeDtypeStruct((N, D), jnp.float32),
            jax.ShapeDtypeStruct((N, D), jnp.float32),
            jax.ShapeDtypeStruct((N,), jnp.float32),
            jax.ShapeDtypeStruct((N,), jnp.float32),
        ],
        mesh=mesh,
        scratch_types=[
            pltpu.VMEM((GC, CB), jnp.int32),
            pltpu.VMEM((GC, CB), jnp.int32),
            pltpu.VMEM((CB, D), jnp.float32),
            pltpu.VMEM((CB, D), jnp.float32),
            pltpu.VMEM((CB,), jnp.float32),
            pltpu.VMEM_SHARED((N, D), jnp.float32),
            pltpu.VMEM_SHARED((N,), jnp.float32),
            pltpu.SemaphoreType.DMA,
        ],
    )


BN = 1000


def _mid_body(y0, y1, c0, c1, w1p, b1p, w1t, b1t, w2p, w2t, z, dinv, mv):
    cnt = c0[...] + c1[...]
    di = 1.0 / jnp.maximum(cnt, 1.0)
    m = cnt * di
    y = (y0[...] + y1[...]) * di
    ap = jnp.dot(y, w1p[...], preferred_element_type=jnp.float32) + m * b1p[...]
    at = jnp.dot(y, w1t[...], preferred_element_type=jnp.float32) + m * b1t[...]
    rp = jnp.maximum(ap, 0.0)
    rt = jnp.maximum(at, 0.0)
    z[...] = (jnp.dot(rt, w2t[...], preferred_element_type=jnp.float32)
              - jnp.dot(rp, w2p[...], preferred_element_type=jnp.float32))
    dinv[...] = di
    mv[...] = m


def _loss_body(u0, u1, dinv, mv, b2p, b2t, out):
    i = pl.program_id(0)
    nb = pl.num_programs(0)
    diff = (u0[...] + u1[...]) * dinv[...] + mv[...] * (b2t[...] - b2p[...])
    part = jnp.sum(diff * diff)
    tot = jnp.where(i == 0, part, out[...] + part)
    out[...] = tot * jnp.where(i == nb - 1, 1.0 / (N * D), 1.0)


def _row_spec(bn, w):
    return pl.BlockSpec((bn, w), lambda i: (i, 0))


def _full_spec(a, b):
    return pl.BlockSpec((a, b), lambda i: (0, 0))


_tc_mid = pl.pallas_call(
    _mid_body,
    grid=(N // BN,),
    in_specs=[
        _row_spec(BN, D), _row_spec(BN, D),
        _row_spec(BN, 1), _row_spec(BN, 1),
        _full_spec(D, D), _full_spec(1, D),
        _full_spec(D, D), _full_spec(1, D),
        _full_spec(D, D), _full_spec(D, D),
    ],
    out_specs=[_row_spec(BN, D), _row_spec(BN, 1), _row_spec(BN, 1)],
    out_shape=[
        jax.ShapeDtypeStruct((N, D), jnp.float32),
        jax.ShapeDtypeStruct((N, 1), jnp.float32),
        jax.ShapeDtypeStruct((N, 1), jnp.float32),
    ],
)

_tc_loss = pl.pallas_call(
    _loss_body,
    grid=(N // BN,),
    in_specs=[
        _row_spec(BN, D), _row_spec(BN, D),
        _row_spec(BN, 1), _row_spec(BN, 1),
        _full_spec(1, D), _full_spec(1, D),
    ],
    out_specs=pl.BlockSpec((1, 1), lambda i: (0, 0)),
    out_shape=jax.ShapeDtypeStruct((1, 1), jnp.float32),
)


def kernel(x, edge_index, W1p, b1p, W2p, b2p, W1t, b1t, W2t, b2t):
    src3 = edge_index[0].reshape(NW, CK, CB)
    dst3 = edge_index[1].reshape(NW, CK, CB)
    zrows = jnp.zeros((N, D), jnp.float32)
    zflat = jnp.zeros((N,), jnp.float32)
    ones = jnp.ones((CB,), jnp.float32)

    y0, y1, c0, c1 = _sc_aggregate()(x, src3, dst3, zrows, zflat, ones)
    z, dinv, mv = _tc_mid(y0, y1, c0.reshape(N, 1), c1.reshape(N, 1),
                          W1p, b1p.reshape(1, D), W1t, b1t.reshape(1, D),
                          W2p, W2t)
    u0, u1, _, _ = _sc_aggregate()(z, src3, dst3, zrows, zflat, ones)
    loss = _tc_loss(u0, u1, dinv, mv, b2p.reshape(1, D), b2t.reshape(1, D))
    return loss.reshape(())
